# baseline (speedup 1.0000x reference)
import functools

import jax
import jax.numpy as jnp
from jax import lax
from jax.experimental import pallas as pl
from jax.experimental.pallas import tpu as pltpu
from jax.experimental.pallas import tpu_sc as plsc

S = 64.0
MARGIN = 0.4

NC = 2
NS = 16
NW = NC * NS
L = 16


def _make_sc_kernel(B, V, chunk):
    rows_per_w = B // NW
    n_chunks = V // chunk
    mesh = plsc.VectorSubcoreMesh(core_axis_name="c", subcore_axis_name="s")

    @functools.partial(
        pl.kernel,
        mesh=mesh,
        out_type=jax.ShapeDtypeStruct((B * V,), jnp.float32),
        scratch_types=[
            pltpu.VMEM((rows_per_w + L,), jnp.int32),
            pltpu.VMEM((chunk,), jnp.float32),
        ],
    )
    def sc_fn(logits_hbm, labels_hbm, out_hbm, lab_v, buf):
        wid = lax.axis_index("s") * NC + lax.axis_index("c")
        base_row = wid * rows_per_w
        pltpu.sync_copy(
            labels_hbm.at[pl.ds(base_row, rows_per_w)],
            lab_v.at[pl.ds(0, rows_per_w)],
        )
        lane_iota = lax.iota(jnp.int32, L)

        @pl.loop(0, rows_per_w)
        def _row(r):
            lab = lab_v[pl.ds(r, L)][0]
            row_off = (base_row + r) * V

            @pl.loop(0, n_chunks)
            def _chunk(k):
                off = k * chunk
                pltpu.sync_copy(logits_hbm.at[pl.ds(row_off + off, chunk)], buf)

                @plsc.parallel_loop(0, chunk // L, unroll=8)
                def _v(i):
                    sl = pl.ds(i * L, L)
                    buf[sl] = buf[sl] * S

                @pl.when((lab >= off) & (lab < off + chunk))
                def _fix():
                    rel = lab - off
                    sl = pl.ds((rel // L) * L, L)
                    delta = jnp.where(lane_iota == rel % L, MARGIN * S, 0.0)
                    buf[sl] = buf[sl] - delta

                pltpu.sync_copy(buf, out_hbm.at[pl.ds(row_off + off, chunk)])

    return sc_fn


def kernel(logits, labels, embeddings):
    B, V = logits.shape
    sc_fn = _make_sc_kernel(B, V, chunk=20000)
    flat = sc_fn(logits.reshape(B * V), labels.astype(jnp.int32))
    return flat.reshape(B, V)


# speedup vs baseline: 1.1125x; 1.1125x over previous
import functools

import jax
import jax.numpy as jnp
from jax import lax
from jax.experimental import pallas as pl
from jax.experimental.pallas import tpu as pltpu
from jax.experimental.pallas import tpu_sc as plsc

S = 64.0
MARGIN = 0.4

NC = 2
NS = 16
NW = NC * NS
L = 16
NBUF = 4


def _make_sc_kernel(B, V, chunk, unroll):
    rows_per_w = B // NW
    ncpr = V // chunk
    total = rows_per_w * ncpr
    mesh = plsc.VectorSubcoreMesh(core_axis_name="c", subcore_axis_name="s")

    @functools.partial(
        pl.kernel,
        mesh=mesh,
        out_type=jax.ShapeDtypeStruct((B * V,), jnp.float32),
        scratch_types=[
            pltpu.VMEM((rows_per_w + L,), jnp.int32),
            pltpu.VMEM((NBUF * chunk,), jnp.float32),
            pltpu.VMEM((NBUF * chunk,), jnp.float32),
            pltpu.SemaphoreType.DMA((NBUF,)),
            pltpu.SemaphoreType.DMA((NBUF,)),
        ],
    )
    def sc_fn(logits_hbm, labels_hbm, out_hbm, lab_v, ibuf, obuf, isem, osem):
        wid = lax.axis_index("s") * NC + lax.axis_index("c")
        base_row = wid * rows_per_w
        pltpu.sync_copy(
            labels_hbm.at[pl.ds(base_row, rows_per_w)],
            lab_v.at[pl.ds(0, rows_per_w)],
        )
        lane_iota = lax.iota(jnp.int32, L)
        hbm_base = base_row * V

        def chunk_off(c):
            return hbm_base + (c // ncpr) * V + (c % ncpr) * chunk

        def start_in(b, c):
            pltpu.async_copy(
                logits_hbm.at[pl.ds(chunk_off(c), chunk)],
                ibuf.at[pl.ds(b * chunk, chunk)],
                isem.at[b],
            )

        def wait(sem_b, dst_slice):
            pltpu.make_async_copy(
                logits_hbm.at[pl.ds(0, chunk)], dst_slice, sem_b
            ).wait()

        for b in range(NBUF):
            start_in(b, b)

        @pl.loop(0, total, step=NBUF)
        def _ring(g):
            for b in range(NBUF):
                c = g + b
                wait(isem.at[b], ibuf.at[pl.ds(b * chunk, chunk)])

                @pl.when(c >= NBUF)
                def _():
                    wait(osem.at[b], obuf.at[pl.ds(b * chunk, chunk)])

                @plsc.parallel_loop(0, chunk // L, unroll=unroll)
                def _v(i):
                    sl = pl.ds(b * chunk + i * L, L)
                    obuf[sl] = ibuf[sl] * S

                lab = lab_v[pl.ds(c // ncpr, L)][0]
                off = (c % ncpr) * chunk

                @pl.when((lab >= off) & (lab < off + chunk))
                def _fix():
                    rel = lab - off
                    sl = pl.ds(b * chunk + (rel // L) * L, L)
                    delta = jnp.where(lane_iota == rel % L, MARGIN * S, 0.0)
                    obuf[sl] = obuf[sl] - delta

                pltpu.async_copy(
                    obuf.at[pl.ds(b * chunk, chunk)],
                    out_hbm.at[pl.ds(chunk_off(c), chunk)],
                    osem.at[b],
                )

                @pl.when(c + NBUF < total)
                def _next():
                    start_in(b, c + NBUF)

        for b in range(NBUF):
            wait(osem.at[b], obuf.at[pl.ds(b * chunk, chunk)])

    return sc_fn


def kernel(logits, labels, embeddings):
    B, V = logits.shape
    sc_fn = _make_sc_kernel(B, V, chunk=10000, unroll=8)
    flat = sc_fn(logits.reshape(B * V), labels.astype(jnp.int32))
    return flat.reshape(B, V)


# speedup vs baseline: 2.1967x; 1.9745x over previous
import functools

import jax
import jax.numpy as jnp
from jax import lax
from jax.experimental import pallas as pl
from jax.experimental.pallas import tpu as pltpu
from jax.experimental.pallas import tpu_sc as plsc

S = 64.0
MARGIN = 0.4

NC = 2
NS = 16
NW = NC * NS
L = 16
NBUF = 4
RS = 8
CC = 1024


def _make_sc_kernel(B, V):
    rows_per_w = B // NW
    slabs = rows_per_w // RS
    ncc = V // CC
    tail = V - ncc * CC
    total = slabs * ncc
    mesh = plsc.VectorSubcoreMesh(core_axis_name="c", subcore_axis_name="s")

    @functools.partial(
        pl.kernel,
        mesh=mesh,
        out_type=jax.ShapeDtypeStruct((B, V), jnp.float32),
        scratch_types=[
            pltpu.VMEM((rows_per_w + L,), jnp.int32),
            pltpu.VMEM((NBUF, RS, CC), jnp.float32),
            pltpu.VMEM((NBUF, RS, CC), jnp.float32),
            pltpu.VMEM((RS, tail), jnp.float32),
            pltpu.SemaphoreType.DMA((NBUF,)),
            pltpu.SemaphoreType.DMA((NBUF,)),
        ],
    )
    def sc_fn(logits_hbm, labels_hbm, out_hbm, lab_v, ibuf, obuf, tbuf,
              isem, osem):
        wid = lax.axis_index("s") * NC + lax.axis_index("c")
        base_row = wid * rows_per_w
        pltpu.sync_copy(
            labels_hbm.at[pl.ds(base_row, rows_per_w)],
            lab_v.at[pl.ds(0, rows_per_w)],
        )
        lane_iota = lax.iota(jnp.int32, L)

        def slab_slice(ref, c):
            s, k = c // ncc, c % ncc
            r0 = pl.multiple_of(base_row + s * RS, RS)
            off = pl.multiple_of(k * CC, 128)
            return ref.at[pl.ds(r0, RS), pl.ds(off, CC)]

        def apply_margin(buf, c_row0, off, width):
            for lr in range(RS):
                lab = lab_v[pl.ds(c_row0 - base_row + lr, L)][0]

                @pl.when((lab >= off) & (lab < off + width))
                def _():
                    rel = lab - off
                    sl = pl.ds((rel // L) * L, L)
                    delta = jnp.where(lane_iota == rel % L, MARGIN * S, 0.0)
                    buf[lr, sl] = buf[lr, sl] - delta

        def start_in(b, c):
            pltpu.async_copy(slab_slice(logits_hbm, c), ibuf.at[b], isem.at[b])

        def wait(sem_b, dst):
            pltpu.make_async_copy(
                logits_hbm.at[pl.ds(0, RS), pl.ds(0, CC)], dst, sem_b
            ).wait()

        for b in range(NBUF):
            start_in(b, b)

        @pl.loop(0, total, step=NBUF)
        def _ring(g):
            for b in range(NBUF):
                c = g + b
                wait(isem.at[b], ibuf.at[b])

                @pl.when(c >= NBUF)
                def _():
                    wait(osem.at[b], obuf.at[b])

                for lr in range(RS):

                    @plsc.parallel_loop(0, CC // L, unroll=8)
                    def _v(i):
                        sl = pl.ds(i * L, L)
                        obuf[b, lr, sl] = ibuf[b, lr, sl] * S

                s = c // ncc
                apply_margin(obuf.at[b], base_row + s * RS, (c % ncc) * CC, CC)
                pltpu.async_copy(obuf.at[b], slab_slice(out_hbm, c), osem.at[b])

                @pl.when(c + NBUF < total)
                def _next():
                    start_in(b, c + NBUF)

        for b in range(NBUF):
            wait(osem.at[b], obuf.at[b])

        toff = ncc * CC

        @pl.loop(0, slabs)
        def _tail(s):
            r0 = pl.multiple_of(base_row + s * RS, RS)
            pltpu.sync_copy(
                logits_hbm.at[pl.ds(r0, RS), pl.ds(toff, tail)], tbuf
            )
            for lr in range(RS):

                @plsc.parallel_loop(0, tail // L, unroll=8)
                def _v(i):
                    sl = pl.ds(i * L, L)
                    tbuf[lr, sl] = tbuf[lr, sl] * S

            apply_margin(tbuf, base_row + s * RS, toff, tail)
            pltpu.sync_copy(
                tbuf, out_hbm.at[pl.ds(r0, RS), pl.ds(toff, tail)]
            )

    return sc_fn


def kernel(logits, labels, embeddings):
    B, V = logits.shape
    sc_fn = _make_sc_kernel(B, V)
    return sc_fn(logits, labels.astype(jnp.int32))


# speedup vs baseline: 7.0965x; 3.2305x over previous
import functools

import jax
import jax.numpy as jnp
from jax import lax
from jax.experimental import pallas as pl
from jax.experimental.pallas import tpu as pltpu
from jax.experimental.pallas import tpu_sc as plsc

S = 64.0
MARGIN = 0.4

NC = 2
NS = 16
NW = NC * NS
L = 16
NBUF = 3
R = 16


def _make_sc_kernel(B, V):
    nchunks = V // R
    tmax = pl.cdiv(nchunks, NW)
    ngroups = B // L
    mesh = plsc.VectorSubcoreMesh(core_axis_name="c", subcore_axis_name="s")

    @functools.partial(
        pl.kernel,
        mesh=mesh,
        out_type=jax.ShapeDtypeStruct((V, B), jnp.float32),
        scratch_types=[
            pltpu.VMEM((B,), jnp.int32),
            pltpu.VMEM((NBUF, R, B), jnp.float32),
            pltpu.VMEM((NBUF, R, B), jnp.float32),
            pltpu.SemaphoreType.DMA((NBUF,)),
            pltpu.SemaphoreType.DMA((NBUF,)),
        ],
    )
    def sc_fn(xt_hbm, labels_hbm, out_hbm, lab_v, ibuf, obuf, isem, osem):
        wid = lax.axis_index("s") * NC + lax.axis_index("c")
        pltpu.sync_copy(labels_hbm, lab_v)
        neg = jnp.full((L,), -MARGIN * S, jnp.float32)
        my_t = jnp.where(wid < nchunks - NW * (tmax - 1), tmax, tmax - 1)

        def start_in(b, t):
            off = pl.multiple_of((wid + t * NW) * R, R)
            pltpu.async_copy(
                xt_hbm.at[pl.ds(off, R), :], ibuf.at[b], isem.at[b]
            )

        def wait(sem_b, dst):
            pltpu.make_async_copy(
                xt_hbm.at[pl.ds(0, R), :], dst, sem_b
            ).wait()

        for b in range(NBUF):
            start_in(b, b)

        @pl.loop(0, pl.cdiv(tmax, NBUF) * NBUF, step=NBUF)
        def _ring(g):
          for b in range(NBUF):
            t = g + b

            @pl.when(t < my_t)
            def _step():
                voff = (wid + t * NW) * R
                wait(isem.at[b], ibuf.at[b])

                @pl.when(t >= NBUF)
                def _():
                    wait(osem.at[b], obuf.at[b])

                @plsc.parallel_loop(0, ngroups, unroll=2)
                def _v(i):
                    sl = pl.ds(i * L, L)
                    labv = lab_v[sl]
                    for lr in range(R):
                        x = ibuf[b, lr, sl] * S
                        obuf[b, lr, sl] = jnp.where(
                            labv == voff + lr, x + neg, x
                        )

                off = pl.multiple_of(voff, R)
                pltpu.async_copy(
                    obuf.at[b], out_hbm.at[pl.ds(off, R), :], osem.at[b]
                )

                @pl.when(t + NBUF < my_t)
                def _next():
                    start_in(b, t + NBUF)

        for b in range(NBUF):
            wait(osem.at[b], obuf.at[b])

    return sc_fn


def kernel(logits, labels, embeddings):
    B, V = logits.shape
    sc_fn = _make_sc_kernel(B, V)
    out_t = sc_fn(logits.T, labels.astype(jnp.int32))
    return out_t.T


# speedup vs baseline: 7.0967x; 1.0000x over previous
import functools

import jax
import jax.numpy as jnp
from jax import lax
from jax.experimental import pallas as pl
from jax.experimental.pallas import tpu as pltpu
from jax.experimental.pallas import tpu_sc as plsc

S = 64.0
MARGIN = 0.4

NC = 2
NS = 16
NW = NC * NS
L = 16
NBUF = 3
R = 16


def _make_sc_kernel(B, V):
    nchunks = V // R
    tmax = pl.cdiv(nchunks, NW)
    ngroups = B // L
    mesh = plsc.VectorSubcoreMesh(core_axis_name="c", subcore_axis_name="s")

    @functools.partial(
        pl.kernel,
        mesh=mesh,
        out_type=jax.ShapeDtypeStruct((V, B), jnp.float32),
        scratch_types=[
            pltpu.VMEM((B,), jnp.int32),
            pltpu.VMEM((NBUF, R, B), jnp.float32),
            pltpu.VMEM((NBUF, R, B), jnp.float32),
            pltpu.SemaphoreType.DMA((NBUF,)),
            pltpu.SemaphoreType.DMA((NBUF,)),
        ],
    )
    def sc_fn(xt_hbm, labels_hbm, out_hbm, lab_v, ibuf, obuf, isem, osem):
        wid = lax.axis_index("s") * NC + lax.axis_index("c")
        pltpu.sync_copy(labels_hbm, lab_v)
        neg = jnp.full((L,), -MARGIN * S, jnp.float32)
        my_t = jnp.where(wid < nchunks - NW * (tmax - 1), tmax, tmax - 1)

        def start_in(b, t):
            off = pl.multiple_of((wid + t * NW) * R, R)
            pltpu.async_copy(
                xt_hbm.at[pl.ds(off, R), :], ibuf.at[b], isem.at[b]
            )

        def wait(sem_b, dst):
            pltpu.make_async_copy(
                xt_hbm.at[pl.ds(0, R), :], dst, sem_b
            ).wait()

        for b in range(NBUF):
            start_in(b, b)

        @pl.loop(0, pl.cdiv(tmax, NBUF) * NBUF, step=NBUF)
        def _ring(g):
          for b in range(NBUF):
            t = g + b

            @pl.when(t < my_t)
            def _step():
                voff = (wid + t * NW) * R
                wait(isem.at[b], ibuf.at[b])

                @pl.when(t >= NBUF)
                def _():
                    wait(osem.at[b], obuf.at[b])

                @plsc.parallel_loop(0, ngroups, unroll=4)
                def _v(i):
                    sl = pl.ds(i * L, L)
                    labv = lab_v[sl]
                    for lr in range(R):
                        x = ibuf[b, lr, sl] * S
                        obuf[b, lr, sl] = jnp.where(
                            labv == voff + lr, x + neg, x
                        )

                off = pl.multiple_of(voff, R)
                pltpu.async_copy(
                    obuf.at[b], out_hbm.at[pl.ds(off, R), :], osem.at[b]
                )

                @pl.when(t + NBUF < my_t)
                def _next():
                    start_in(b, t + NBUF)

        for b in range(NBUF):
            wait(osem.at[b], obuf.at[b])

    return sc_fn


def kernel(logits, labels, embeddings):
    B, V = logits.shape
    sc_fn = _make_sc_kernel(B, V)
    out_t = sc_fn(logits.T, labels.astype(jnp.int32))
    return out_t.T


# speedup vs baseline: 7.2964x; 1.0281x over previous
import functools

import jax
import jax.numpy as jnp
from jax import lax
from jax.experimental import pallas as pl
from jax.experimental.pallas import tpu as pltpu
from jax.experimental.pallas import tpu_sc as plsc

S = 64.0
MARGIN = 0.4

NC = 2
NS = 16
NW = NC * NS
L = 16
NBUF = 6
R = 8


def _make_sc_kernel(B, V):
    nchunks = V // R
    tmax = pl.cdiv(nchunks, NW)
    ngroups = B // L
    mesh = plsc.VectorSubcoreMesh(core_axis_name="c", subcore_axis_name="s")

    @functools.partial(
        pl.kernel,
        mesh=mesh,
        out_type=jax.ShapeDtypeStruct((V, B), jnp.float32),
        scratch_types=[
            pltpu.VMEM((B,), jnp.int32),
            pltpu.VMEM((NBUF, R, B), jnp.float32),
            pltpu.VMEM((NBUF, R, B), jnp.float32),
            pltpu.SemaphoreType.DMA((NBUF,)),
            pltpu.SemaphoreType.DMA((NBUF,)),
        ],
    )
    def sc_fn(xt_hbm, labels_hbm, out_hbm, lab_v, ibuf, obuf, isem, osem):
        wid = lax.axis_index("s") * NC + lax.axis_index("c")
        pltpu.sync_copy(labels_hbm, lab_v)
        neg = jnp.full((L,), -MARGIN * S, jnp.float32)
        my_t = jnp.where(wid < nchunks - NW * (tmax - 1), tmax, tmax - 1)

        def start_in(b, t):
            off = pl.multiple_of((wid + t * NW) * R, R)
            pltpu.async_copy(
                xt_hbm.at[pl.ds(off, R), :], ibuf.at[b], isem.at[b]
            )

        def wait(sem_b, dst):
            pltpu.make_async_copy(
                xt_hbm.at[pl.ds(0, R), :], dst, sem_b
            ).wait()

        for b in range(NBUF):
            start_in(b, b)

        @pl.loop(0, pl.cdiv(tmax, NBUF) * NBUF, step=NBUF)
        def _ring(g):
          for b in range(NBUF):
            t = g + b

            @pl.when(t < my_t)
            def _step():
                voff = (wid + t * NW) * R
                wait(isem.at[b], ibuf.at[b])

                @pl.when(t >= NBUF)
                def _():
                    wait(osem.at[b], obuf.at[b])

                @plsc.parallel_loop(0, ngroups, unroll=4)
                def _v(i):
                    sl = pl.ds(i * L, L)
                    labv = lab_v[sl]
                    for lr in range(R):
                        x = ibuf[b, lr, sl] * S
                        obuf[b, lr, sl] = jnp.where(
                            labv == voff + lr, x + neg, x
                        )

                off = pl.multiple_of(voff, R)
                pltpu.async_copy(
                    obuf.at[b], out_hbm.at[pl.ds(off, R), :], osem.at[b]
                )

                @pl.when(t + NBUF < my_t)
                def _next():
                    start_in(b, t + NBUF)

        for b in range(NBUF):
            wait(osem.at[b], obuf.at[b])

    return sc_fn


def kernel(logits, labels, embeddings):
    B, V = logits.shape
    sc_fn = _make_sc_kernel(B, V)
    out_t = sc_fn(logits.T, labels.astype(jnp.int32))
    return out_t.T
